# baseline (speedup 1.0000x reference)
import functools

import jax
import jax.numpy as jnp
from jax import lax
from jax.experimental import pallas as pl
from jax.experimental.pallas import tpu as pltpu
from jax.experimental.pallas import tpu_sc as plsc

NC = 2
NS = 16
NW = NC * NS
BATCH = 16384
BPW = BATCH // NW
NCHUNK = 4
CHUNK = BPW // NCHUNK
BLK = 65536


def _collapse_body(W1r, b1r, W2r, b2r, W3r, b3r, outr):
    w32 = jnp.dot(W3r[...], W2r[...], preferred_element_type=jnp.float32,
                  precision=lax.Precision.HIGHEST)
    w100 = jnp.dot(w32, W1r[...], preferred_element_type=jnp.float32,
                   precision=lax.Precision.HIGHEST)
    c = jnp.sum(w32 * b1r[...]) + jnp.sum(W3r[...] * b2r[...]) + b3r[0, 0]
    outr[...] = jnp.concatenate(
        [w100, jnp.reshape(c, (1, 1)), jnp.zeros((1, 27), jnp.float32)], axis=1)


def _collapse(W1, b1, W2, b2, W3, b3):
    return pl.pallas_call(
        _collapse_body,
        out_shape=jax.ShapeDtypeStruct((1, 128), jnp.float32),
    )(W1, b1.reshape(1, 40), W2, b2.reshape(1, 20), W3, b3.reshape(1, 1))


def _matvec_body(w_ref, tab_ref, bias_ref, out_ref):
    c = 512
    w = w_ref[...]
    bias = bias_ref[0, 0]

    def step(j, carry):
        sub = tab_ref[:, pl.ds(j * c, c)] * w
        out_ref[pl.ds(j * c, c)] = jnp.sum(sub, axis=0) + bias
        return carry

    lax.fori_loop(0, BLK // c, step, 0)


def _matvec(tab_t, w, bias):
    n = tab_t.shape[1]
    grid = (n + BLK - 1) // BLK
    out = pl.pallas_call(
        _matvec_body,
        grid=(grid,),
        in_specs=[
            pl.BlockSpec((50, 1), lambda i: (0, 0)),
            pl.BlockSpec((50, BLK), lambda i: (0, i)),
            pl.BlockSpec((1, 1), lambda i: (0, 0)),
        ],
        out_specs=pl.BlockSpec((BLK,), lambda i: (i,)),
        out_shape=jax.ShapeDtypeStruct((n,), jnp.float32),
    )(w, tab_t, bias)
    return out


def _sc_body(uidx_hbm, midx_hbm, udot_hbm, mdot_hbm, out_hbm,
             uidx_v, midx_v, uval_v, mval_v, sem):
    wid = lax.axis_index("s") * NC + lax.axis_index("c")
    for j in range(NCHUNK):
        pltpu.sync_copy(uidx_hbm.at[wid, pl.ds(j * CHUNK, CHUNK)], uidx_v.at[j])
        pltpu.sync_copy(midx_hbm.at[wid, pl.ds(j * CHUNK, CHUNK)], midx_v.at[j])
    copies = []
    for j in range(NCHUNK):
        copies.append(pltpu.async_copy(udot_hbm.at[uidx_v.at[j]], uval_v.at[j], sem))
        copies.append(pltpu.async_copy(mdot_hbm.at[midx_v.at[j]], mval_v.at[j], sem))
    for cp in copies:
        cp.wait()
    for j in range(NCHUNK):
        for v in range(CHUNK // 16):
            s = (uval_v[j, pl.ds(v * 16, 16)] + mval_v[j, pl.ds(v * 16, 16)])
            uval_v[j, pl.ds(v * 16, 16)] = s
        pltpu.sync_copy(uval_v.at[j], out_hbm.at[wid, pl.ds(j * CHUNK, CHUNK)])


_sc_kernel = functools.partial(
    pl.kernel,
    out_type=jax.ShapeDtypeStruct((NW, BPW), jnp.float32),
    mesh=plsc.VectorSubcoreMesh(core_axis_name="c", subcore_axis_name="s",
                                num_cores=NC, num_subcores=NS),
    compiler_params=pltpu.CompilerParams(needs_layout_passes=False,
                                         use_tc_tiling_on_sc=False),
    scratch_types=[
        pltpu.VMEM((NCHUNK, CHUNK), jnp.int32),
        pltpu.VMEM((NCHUNK, CHUNK), jnp.int32),
        pltpu.VMEM((NCHUNK, CHUNK), jnp.float32),
        pltpu.VMEM((NCHUNK, CHUNK), jnp.float32),
        pltpu.SemaphoreType.DMA,
    ],
)(_sc_body)


def kernel(user, movie, user_table, movie_table, W1, b1, W2, b2, W3, b3):
    wf = _collapse(W1, b1, W2, b2, W3, b3)
    zero = jnp.zeros((1, 1), jnp.float32)
    udot = _matvec(user_table.T, wf[0, 0:50].reshape(50, 1), zero)
    mdot = _matvec(movie_table.T, wf[0, 50:100].reshape(50, 1), wf[:, 100:101])
    uidx = user.astype(jnp.int32).reshape(NW, BPW)
    midx = movie.astype(jnp.int32).reshape(NW, BPW)
    out = _sc_kernel(uidx, midx, udot, mdot)
    return out.reshape(BATCH, 1)


# speedup vs baseline: 2.0749x; 2.0749x over previous
import functools

import jax
import jax.numpy as jnp
from jax import lax
from jax.experimental import pallas as pl
from jax.experimental.pallas import tpu as pltpu
from jax.experimental.pallas import tpu_sc as plsc

NC = 2
NS = 16
NW = NC * NS
BATCH = 16384
BPW = BATCH // NW
NCHUNK = 4
CHUNK = BPW // NCHUNK
BLK = 98304


def _collapse_body(W1r, b1r, W2r, b2r, W3r, b3r, outr):
    w32 = jnp.dot(W3r[...], W2r[...], preferred_element_type=jnp.float32,
                  precision=lax.Precision.HIGHEST)
    w100 = jnp.dot(w32, W1r[...], preferred_element_type=jnp.float32,
                   precision=lax.Precision.HIGHEST)
    c = jnp.sum(w32 * b1r[...]) + jnp.sum(W3r[...] * b2r[...]) + b3r[0, 0]
    outr[...] = jnp.concatenate(
        [w100, jnp.reshape(c, (1, 1)), jnp.zeros((1, 27), jnp.float32)], axis=1)


def _collapse(W1, b1, W2, b2, W3, b3):
    return pl.pallas_call(
        _collapse_body,
        out_shape=jax.ShapeDtypeStruct((1, 128), jnp.float32),
    )(W1, b1.reshape(1, 40), W2, b2.reshape(1, 20), W3, b3.reshape(1, 1))


def _matvec_body(w_ref, tab_ref, bias_ref, out_ref):
    prod = tab_ref[...] * w_ref[...]
    out_ref[...] = jnp.sum(prod, axis=0) + bias_ref[0, 0]


def _matvec(tab_t, w, bias):
    n = tab_t.shape[1]
    grid = (n + BLK - 1) // BLK
    out = pl.pallas_call(
        _matvec_body,
        grid=(grid,),
        in_specs=[
            pl.BlockSpec((50, 1), lambda i: (0, 0)),
            pl.BlockSpec((50, BLK), lambda i: (0, i)),
            pl.BlockSpec((1, 1), lambda i: (0, 0)),
        ],
        out_specs=pl.BlockSpec((BLK,), lambda i: (i,)),
        out_shape=jax.ShapeDtypeStruct((n,), jnp.float32),
    )(w, tab_t, bias)
    return out


def _sc_body(uidx_hbm, midx_hbm, udot_hbm, mdot_hbm, out_hbm,
             uidx_v, midx_v, uval_v, mval_v, sem):
    wid = lax.axis_index("s") * NC + lax.axis_index("c")
    for j in range(NCHUNK):
        pltpu.sync_copy(uidx_hbm.at[wid, pl.ds(j * CHUNK, CHUNK)], uidx_v.at[j])
        pltpu.sync_copy(midx_hbm.at[wid, pl.ds(j * CHUNK, CHUNK)], midx_v.at[j])
    copies = []
    for j in range(NCHUNK):
        copies.append(pltpu.async_copy(udot_hbm.at[uidx_v.at[j]], uval_v.at[j], sem))
        copies.append(pltpu.async_copy(mdot_hbm.at[midx_v.at[j]], mval_v.at[j], sem))
    for cp in copies:
        cp.wait()
    for j in range(NCHUNK):
        for v in range(CHUNK // 16):
            s = (uval_v[j, pl.ds(v * 16, 16)] + mval_v[j, pl.ds(v * 16, 16)])
            uval_v[j, pl.ds(v * 16, 16)] = s
        pltpu.sync_copy(uval_v.at[j], out_hbm.at[wid, pl.ds(j * CHUNK, CHUNK)])


_sc_kernel = functools.partial(
    pl.kernel,
    out_type=jax.ShapeDtypeStruct((NW, BPW), jnp.float32),
    mesh=plsc.VectorSubcoreMesh(core_axis_name="c", subcore_axis_name="s",
                                num_cores=NC, num_subcores=NS),
    compiler_params=pltpu.CompilerParams(needs_layout_passes=False,
                                         use_tc_tiling_on_sc=False),
    scratch_types=[
        pltpu.VMEM((NCHUNK, CHUNK), jnp.int32),
        pltpu.VMEM((NCHUNK, CHUNK), jnp.int32),
        pltpu.VMEM((NCHUNK, CHUNK), jnp.float32),
        pltpu.VMEM((NCHUNK, CHUNK), jnp.float32),
        pltpu.SemaphoreType.DMA,
    ],
)(_sc_body)


def kernel(user, movie, user_table, movie_table, W1, b1, W2, b2, W3, b3):
    wf = _collapse(W1, b1, W2, b2, W3, b3)
    zero = jnp.zeros((1, 1), jnp.float32)
    udot = _matvec(user_table.T, wf[0, 0:50].reshape(50, 1), zero)
    mdot = _matvec(movie_table.T, wf[0, 50:100].reshape(50, 1), wf[:, 100:101])
    uidx = user.astype(jnp.int32).reshape(NW, BPW)
    midx = movie.astype(jnp.int32).reshape(NW, BPW)
    out = _sc_kernel(uidx, midx, udot, mdot)
    return out.reshape(BATCH, 1)


# speedup vs baseline: 2.2000x; 1.0603x over previous
import functools

import jax
import jax.numpy as jnp
from jax import lax
from jax.experimental import pallas as pl
from jax.experimental.pallas import tpu as pltpu
from jax.experimental.pallas import tpu_sc as plsc

NC = 2
NS = 16
NW = NC * NS
BATCH = 16384
BPW = BATCH // NW
NCHUNK = 4
CHUNK = BPW // NCHUNK
BLK = 65536


def _collapse_body(W1r, b1r, W2r, b2r, W3r, b3r, outr):
    w32 = jnp.dot(W3r[...], W2r[...], preferred_element_type=jnp.float32,
                  precision=lax.Precision.HIGHEST)
    w100 = jnp.dot(w32, W1r[...], preferred_element_type=jnp.float32,
                   precision=lax.Precision.HIGHEST)
    c = jnp.sum(w32 * b1r[...]) + jnp.sum(W3r[...] * b2r[...]) + b3r[0, 0]
    outr[...] = jnp.concatenate(
        [w100, jnp.reshape(c, (1, 1)), jnp.zeros((1, 27), jnp.float32)], axis=1)


def _collapse(W1, b1, W2, b2, W3, b3):
    return pl.pallas_call(
        _collapse_body,
        out_shape=jax.ShapeDtypeStruct((1, 128), jnp.float32),
    )(W1, b1.reshape(1, 40), W2, b2.reshape(1, 20), W3, b3.reshape(1, 1))


def _matvec_body(w_ref, tab_ref, bias_ref, out_ref):
    prod = tab_ref[...] * w_ref[...]
    out_ref[...] = jnp.sum(prod, axis=0) + bias_ref[0, 0]


def _matvec(tab_t, w, bias):
    n = tab_t.shape[1]
    grid = (n + BLK - 1) // BLK
    out = pl.pallas_call(
        _matvec_body,
        grid=(grid,),
        in_specs=[
            pl.BlockSpec((50, 1), lambda i: (0, 0)),
            pl.BlockSpec((50, BLK), lambda i: (0, i)),
            pl.BlockSpec((1, 1), lambda i: (0, 0)),
        ],
        out_specs=pl.BlockSpec((BLK,), lambda i: (i,)),
        out_shape=jax.ShapeDtypeStruct((n,), jnp.float32),
    )(w, tab_t, bias)
    return out


def _sc_body(uidx_hbm, midx_hbm, udot_hbm, mdot_hbm, out_hbm,
             uidx_v, midx_v, uval_v, mval_v, sem):
    wid = lax.axis_index("s") * NC + lax.axis_index("c")
    for j in range(NCHUNK):
        pltpu.sync_copy(uidx_hbm.at[wid, pl.ds(j * CHUNK, CHUNK)], uidx_v.at[j])
        pltpu.sync_copy(midx_hbm.at[wid, pl.ds(j * CHUNK, CHUNK)], midx_v.at[j])
    copies = []
    for j in range(NCHUNK):
        copies.append(pltpu.async_copy(udot_hbm.at[uidx_v.at[j]], uval_v.at[j], sem))
        copies.append(pltpu.async_copy(mdot_hbm.at[midx_v.at[j]], mval_v.at[j], sem))
    for cp in copies:
        cp.wait()
    for j in range(NCHUNK):
        for v in range(CHUNK // 16):
            s = (uval_v[j, pl.ds(v * 16, 16)] + mval_v[j, pl.ds(v * 16, 16)])
            uval_v[j, pl.ds(v * 16, 16)] = s
        pltpu.sync_copy(uval_v.at[j], out_hbm.at[wid, pl.ds(j * CHUNK, CHUNK)])


_sc_kernel = functools.partial(
    pl.kernel,
    out_type=jax.ShapeDtypeStruct((NW, BPW), jnp.float32),
    mesh=plsc.VectorSubcoreMesh(core_axis_name="c", subcore_axis_name="s",
                                num_cores=NC, num_subcores=NS),
    compiler_params=pltpu.CompilerParams(needs_layout_passes=False,
                                         use_tc_tiling_on_sc=False),
    scratch_types=[
        pltpu.VMEM((NCHUNK, CHUNK), jnp.int32),
        pltpu.VMEM((NCHUNK, CHUNK), jnp.int32),
        pltpu.VMEM((NCHUNK, CHUNK), jnp.float32),
        pltpu.VMEM((NCHUNK, CHUNK), jnp.float32),
        pltpu.SemaphoreType.DMA,
    ],
)(_sc_body)


def kernel(user, movie, user_table, movie_table, W1, b1, W2, b2, W3, b3):
    wf = _collapse(W1, b1, W2, b2, W3, b3)
    zero = jnp.zeros((1, 1), jnp.float32)
    udot = _matvec(user_table.T, wf[0, 0:50].reshape(50, 1), zero)
    mdot = _matvec(movie_table.T, wf[0, 50:100].reshape(50, 1), wf[:, 100:101])
    uidx = user.astype(jnp.int32).reshape(NW, BPW)
    midx = movie.astype(jnp.int32).reshape(NW, BPW)
    out = _sc_kernel(uidx, midx, udot, mdot)
    return out.reshape(BATCH, 1)
